# Initial kernel scaffold; baseline (speedup 1.0000x reference)
import jax
import jax.numpy as jnp
from jax.experimental import pallas as pl


def kernel(events, w1, b1, w2, b2, w3, b3):
    raise NotImplementedError("write your pallas kernel here")


# speedup vs baseline: 14.2956x; 14.2956x over previous
import functools

import jax
import jax.numpy as jnp
from jax import lax
from jax.experimental import pallas as pl
from jax.experimental.pallas import tpu as pltpu
from jax.experimental.pallas import tpu_sc as plsc

_B, _N = 4, 49152
_H, _W = 180, 240
_S = 48
_SIDX, _ENDBIAS = 3, 10
_EIDX = _S - _ENDBIAS
_NSEG = _EIDX - _SIDX
_SEGLEN = _N // _S
_PAD = 256
_NGRP = 24
_HW = _H * _W
_EV = _NSEG * _SEGLEN
_CH = _EV // 8
_ROWS = _EV // 8 // 128
_ZCHUNK = 2 * _HW // 16
_ZBUF = 5408


def _hist_body(coords_hbm, out_hbm, coords_v, hist_v):
    wid = lax.axis_index("c") * 16 + lax.axis_index("s")

    @pl.when(wid < _NGRP)
    def _():
        pltpu.sync_copy(coords_hbm.at[wid], coords_v)
        zeros16 = jnp.zeros((16,), jnp.float32)

        def zero_body(i, carry):
            hist_v[pl.ds(i * 16, 16)] = zeros16
            return carry

        lax.fori_loop(0, _PAD, zero_body, 0)

        lane_base = lax.iota(jnp.int32, 16) * _PAD
        ones16 = jnp.ones((16,), jnp.float32)

        def ev_body(e, carry):
            v = coords_v[pl.ds(e * 16, 16)]
            plsc.addupdate_scatter(hist_v, [v + lane_base], ones16)
            return carry

        lax.fori_loop(0, _SEGLEN, ev_body, 0)
        pltpu.sync_copy(hist_v, out_hbm.at[wid])


@functools.lru_cache(maxsize=None)
def _get_hist_kernel():
    return pl.kernel(
        _hist_body,
        out_type=jax.ShapeDtypeStruct((_NGRP, 16 * _PAD), jnp.float32),
        mesh=plsc.VectorSubcoreMesh(core_axis_name="c", subcore_axis_name="s"),
        scratch_types=[
            pltpu.VMEM((_SEGLEN * 16,), jnp.int32),
            pltpu.VMEM((16 * _PAD,), jnp.float32),
        ],
        compiler_params=pltpu.CompilerParams(needs_layout_passes=False),
    )


def _scatter_body(idx_hbm, vals_hbm, out_hbm, idx_v, vals_v, zbuf_v, acc_s):
    c = lax.axis_index("c")
    s = lax.axis_index("s")

    zeros16 = jnp.zeros((16,), jnp.float32)

    def zero_body(i, carry):
        zbuf_v[pl.ds(i * 16, 16)] = zeros16
        return carry

    lax.fori_loop(0, _ZBUF // 16, zero_body, 0)
    pltpu.sync_copy(zbuf_v.at[pl.ds(0, _ZCHUNK)],
                    acc_s.at[pl.ds(s * _ZCHUNK, _ZCHUNK)])
    plsc.subcore_barrier()

    b = c * 2 + s // 8
    chunk = s % 8
    pltpu.sync_copy(idx_hbm.at[b, chunk], idx_v)
    pltpu.sync_copy(vals_hbm.at[b, chunk], vals_v)

    def row_body(j, carry):
        pltpu.sync_copy(vals_v.at[j], acc_s.at[idx_v.at[j]], add=True)
        return carry

    lax.fori_loop(0, _ROWS, row_body, 0)
    plsc.subcore_barrier()

    pltpu.sync_copy(acc_s.at[pl.ds(s * _ZCHUNK, _ZCHUNK)],
                    zbuf_v.at[pl.ds(0, _ZCHUNK)])
    pltpu.sync_copy(zbuf_v.at[pl.ds(0, _ZCHUNK)],
                    out_hbm.at[pl.ds(c * 2 * _HW + s * _ZCHUNK, _ZCHUNK)])


@functools.lru_cache(maxsize=None)
def _get_scatter_kernel():
    return pl.kernel(
        _scatter_body,
        out_type=jax.ShapeDtypeStruct((_B * _HW,), jnp.float32),
        mesh=plsc.VectorSubcoreMesh(core_axis_name="c", subcore_axis_name="s"),
        scratch_types=[
            pltpu.VMEM((_ROWS, 128), jnp.int32),
            pltpu.VMEM((_ROWS, 128), jnp.float32),
            pltpu.VMEM((_ZBUF,), jnp.float32),
            pltpu.VMEM_SHARED((2 * _HW,), jnp.float32),
        ],
        compiler_params=pltpu.CompilerParams(needs_layout_passes=False),
    )


def _leaky(x):
    return jnp.where(x >= 0, x, 0.1 * x)


def _sortnet10(vs):
    vs = list(vs)
    for r in range(10):
        for i in range(r % 2, 9, 2):
            lo = jnp.minimum(vs[i], vs[i + 1])
            hi = jnp.maximum(vs[i], vs[i + 1])
            vs[i], vs[i + 1] = lo, hi
    return vs


def _median10(vs):
    sv = _sortnet10(vs)
    return 0.5 * (sv[4] + sv[5])


def _band(n):
    r = lax.broadcasted_iota(jnp.int32, (n, n), 0)
    c = lax.broadcasted_iota(jnp.int32, (n, n), 1)
    return jnp.where(jnp.abs(r - c) <= 2, 1.0, 0.0).astype(jnp.float32)


def _profile(hist, dim, nb):
    a = hist[:, :dim]
    mu = jnp.sum(a) / float(nb)
    var = jnp.sum((a - mu) ** 2) / float(nb - 1)
    sd = jnp.sqrt(var)
    a = jnp.clip(a, 0.0, mu + 3.0 * sd)
    a = a.astype(jnp.bfloat16).astype(jnp.float32)
    k = jnp.asarray(0.04, jnp.bfloat16).astype(jnp.float32)
    blur = k * jnp.dot(_band(_S), jnp.dot(a, _band(dim),
                                          preferred_element_type=jnp.float32, precision=lax.Precision.HIGHEST),
                       preferred_element_type=jnp.float32, precision=lax.Precision.HIGHEST)
    pos = lax.broadcasted_iota(jnp.int32, (_S, dim), 1).astype(jnp.float32)
    m = jnp.sum(blur * pos, axis=1, keepdims=True) / float(_SEGLEN)
    start = m[_SIDX:_SIDX + 1, :]
    aligned = (m - start) - (float(dim // 2) - start)
    return m, jnp.round(aligned).astype(jnp.int32)


def _outlier0(windows):
    med = _median10(windows)
    diffs = [jnp.abs(w - med) for w in windows]
    mad = _median10(diffs)
    safe = jnp.where(mad == 0.0, 1.0, mad)
    flagged = jnp.where((0.6745 * diffs[0] / safe) > 2.0, 1.0, 0.0)
    return jnp.where(mad == 0.0, 0.0, flagged)


def _dense_body(hx_ref, hy_ref, t_ref, x_ref, y_ref, tl_ref,
                w1_ref, b1_ref, w2_ref, b2_ref, w3_ref, b3_ref,
                idx_ref, vals_ref):
    mx, ax = _profile(hx_ref[0], _W, _S * _W)
    my, ay = _profile(hy_ref[0], _H, _S * _H)

    wx = [mx[_SIDX + o:_EIDX + o, :] for o in range(10)]
    wy = [my[_SIDX + o:_EIDX + o, :] for o in range(10)]
    maskc = (1.0 - _outlier0(wx)) * (1.0 - _outlier0(wy))

    th = t_ref[0] / tl_ref[0]
    ev0 = pl.program_id(1) * _CH
    seg = (ev0 + lax.broadcasted_iota(jnp.int32, (_CH, 1), 0)) // _SEGLEN
    segoff = (seg.astype(jnp.float32) + float(_SIDX)) / float(_EIDX - 1)
    tin = th - segoff

    h1 = _leaky(tin * w1_ref[...] + b1_ref[...])
    h2 = _leaky(jnp.dot(h1.astype(jnp.bfloat16), w2_ref[...].astype(jnp.bfloat16),
                        preferred_element_type=jnp.float32)
                + b2_ref[...])
    v = jnp.dot(h2.astype(jnp.bfloat16), w3_ref[...].astype(jnp.bfloat16),
                preferred_element_type=jnp.float32) + b3_ref[...]

    onehot = (seg == lax.broadcasted_iota(jnp.int32, (1, _NSEG), 1)) \
        .astype(jnp.float32)
    maskrep = jnp.dot(onehot, maskc, preferred_element_type=jnp.float32, precision=lax.Precision.HIGHEST)
    vals = th * v * maskrep
    vals_ref[...] = vals.reshape(1, _CH, 1)

    def _coord(ref, aligned, hi):
        al = jnp.dot(onehot, aligned[_SIDX:_EIDX, :].astype(jnp.float32),
                     preferred_element_type=jnp.float32, precision=lax.Precision.HIGHEST).astype(jnp.int32)
        return jnp.clip(ref[0].astype(jnp.int32) - al, 0, hi - 1)

    xi = _coord(x_ref, ax, _W)
    yi = _coord(y_ref, ay, _H)
    slot = lax.rem(pl.program_id(0), 2)
    idx_ref[...] = (xi + _W * yi + slot * _HW).reshape(1, _CH, 1)


def _dense_call(hx, hy, t, x, y, w1, b1, w2, b2, w3, b3):
    full = lambda shape: pl.BlockSpec(shape, lambda b, c: (0,) * len(shape))
    seg_slice = lambda a: a[:, _SIDX * _SEGLEN:_EIDX * _SEGLEN].reshape(_B, _EV, 1)
    ev_spec = pl.BlockSpec((1, _CH, 1), lambda b, c: (b, c, 0))
    return pl.pallas_call(
        _dense_body,
        grid=(_B, _EV // _CH),
        in_specs=[
            pl.BlockSpec((1, _S, _PAD), lambda b, c: (b, 0, 0)),
            pl.BlockSpec((1, _S, _PAD), lambda b, c: (b, 0, 0)),
            ev_spec, ev_spec, ev_spec,
            pl.BlockSpec((1, 1, 1), lambda b, c: (b, 0, 0)),
            full((1, 30)), full((1, 30)), full((30, 30)),
            full((1, 30)), full((30, 1)), full((1, 1)),
        ],
        out_specs=[ev_spec, ev_spec],
        out_shape=[
            jax.ShapeDtypeStruct((_B, _EV, 1), jnp.int32),
            jax.ShapeDtypeStruct((_B, _EV, 1), jnp.float32),
        ],
    )(hx, hy, seg_slice(t), seg_slice(x), seg_slice(y),
      t[:, _N - 1:].reshape(_B, 1, 1), w1, b1, w2, b2, w3, b3)


def kernel(events, w1, b1, w2, b2, w3, b3):
    x = events[..., 0]
    y = events[..., 1]
    t = events[..., 2]

    def _group(coord):
        g = coord.astype(jnp.int32).reshape(_B * _S // 16, 16, _SEGLEN)
        return g.transpose(0, 2, 1).reshape(_B * _S // 16, 16 * _SEGLEN)

    coords = jnp.concatenate([_group(x), _group(y)], axis=0)
    hists = _get_hist_kernel()(coords)
    hists = hists.reshape(2, _B, _S, _PAD)

    idx, vals = _dense_call(hists[0], hists[1], t, x, y,
                            w1, b1.reshape(1, 30), w2, b2.reshape(1, 30),
                            w3, b3.reshape(1, 1))

    idx4 = idx.reshape(_B, 8, _ROWS, 128)
    vals4 = vals.reshape(_B, 8, _ROWS, 128)
    out = _get_scatter_kernel()(idx4, vals4)
    return out.reshape(_B, _H, _W)


# speedup vs baseline: 15.1841x; 1.0621x over previous
import functools

import jax
import jax.numpy as jnp
from jax import lax
from jax.experimental import pallas as pl
from jax.experimental.pallas import tpu as pltpu
from jax.experimental.pallas import tpu_sc as plsc

_B, _N = 4, 49152
_H, _W = 180, 240
_S = 48
_SIDX, _ENDBIAS = 3, 10
_EIDX = _S - _ENDBIAS
_NSEG = _EIDX - _SIDX
_SEGLEN = _N // _S
_PAD = 256
_NGRP = 24
_HW = _H * _W
_EV = _NSEG * _SEGLEN
_CH = _EV // 8
_ROWS = _EV // 8 // 128
_ZCHUNK = 2 * _HW // 16
_ZBUF = 5408


def _hist_body(coords_hbm, out_hbm, coords_v, hist_v):
    wid = lax.axis_index("c") * 16 + lax.axis_index("s")

    @pl.when(wid < _NGRP)
    def _():
        pltpu.sync_copy(coords_hbm.at[wid], coords_v)
        zeros16 = jnp.zeros((16,), jnp.float32)

        def zero_body(i, carry):
            hist_v[pl.ds(i * 16, 16)] = zeros16
            return carry

        lax.fori_loop(0, _PAD, zero_body, 0)

        lane_base = lax.iota(jnp.int32, 16) * _PAD
        ones16 = jnp.ones((16,), jnp.float32)

        def ev_body(e, carry):
            v = coords_v[pl.ds(e * 16, 16)]
            plsc.addupdate_scatter(hist_v, [v + lane_base], ones16)
            return carry

        lax.fori_loop(0, _SEGLEN, ev_body, 0)
        pltpu.sync_copy(hist_v, out_hbm.at[wid])


@functools.lru_cache(maxsize=None)
def _get_hist_kernel():
    return pl.kernel(
        _hist_body,
        out_type=jax.ShapeDtypeStruct((_NGRP, 16 * _PAD), jnp.float32),
        mesh=plsc.VectorSubcoreMesh(core_axis_name="c", subcore_axis_name="s"),
        scratch_types=[
            pltpu.VMEM((_SEGLEN * 16,), jnp.int32),
            pltpu.VMEM((16 * _PAD,), jnp.float32),
        ],
        compiler_params=pltpu.CompilerParams(needs_layout_passes=False),
    )


def _scatter_body(idx_hbm, vals_hbm, out_hbm, idx_v, vals_v, zbuf_v, acc_s):
    c = lax.axis_index("c")
    s = lax.axis_index("s")

    zeros16 = jnp.zeros((16,), jnp.float32)

    def zero_body(i, carry):
        zbuf_v[pl.ds(i * 16, 16)] = zeros16
        return carry

    lax.fori_loop(0, _ZBUF // 16, zero_body, 0)
    pltpu.sync_copy(zbuf_v.at[pl.ds(0, _ZCHUNK)],
                    acc_s.at[pl.ds(s * _ZCHUNK, _ZCHUNK)])
    plsc.subcore_barrier()

    b = c * 2 + s // 8
    chunk = s % 8
    pltpu.sync_copy(idx_hbm.at[b, chunk], idx_v)
    pltpu.sync_copy(vals_hbm.at[b, chunk], vals_v)

    def row_body(j, carry):
        pltpu.sync_copy(vals_v.at[j], acc_s.at[idx_v.at[j]], add=True)
        return carry

    lax.fori_loop(0, _ROWS, row_body, 0)
    plsc.subcore_barrier()

    pltpu.sync_copy(acc_s.at[pl.ds(s * _ZCHUNK, _ZCHUNK)],
                    zbuf_v.at[pl.ds(0, _ZCHUNK)])
    pltpu.sync_copy(zbuf_v.at[pl.ds(0, _ZCHUNK)],
                    out_hbm.at[pl.ds(c * 2 * _HW + s * _ZCHUNK, _ZCHUNK)])


@functools.lru_cache(maxsize=None)
def _get_scatter_kernel():
    return pl.kernel(
        _scatter_body,
        out_type=jax.ShapeDtypeStruct((_B * _HW,), jnp.float32),
        mesh=plsc.VectorSubcoreMesh(core_axis_name="c", subcore_axis_name="s"),
        scratch_types=[
            pltpu.VMEM((_ROWS, 128), jnp.int32),
            pltpu.VMEM((_ROWS, 128), jnp.float32),
            pltpu.VMEM((_ZBUF,), jnp.float32),
            pltpu.VMEM_SHARED((2 * _HW,), jnp.float32),
        ],
        compiler_params=pltpu.CompilerParams(needs_layout_passes=False),
    )


def _leaky(x):
    return jnp.where(x >= 0, x, 0.1 * x)


def _sortnet10(vs):
    vs = list(vs)
    for r in range(10):
        for i in range(r % 2, 9, 2):
            lo = jnp.minimum(vs[i], vs[i + 1])
            hi = jnp.maximum(vs[i], vs[i + 1])
            vs[i], vs[i + 1] = lo, hi
    return vs


def _median10(vs):
    sv = _sortnet10(vs)
    return 0.5 * (sv[4] + sv[5])


def _band(n):
    r = lax.broadcasted_iota(jnp.int32, (n, n), 0)
    c = lax.broadcasted_iota(jnp.int32, (n, n), 1)
    return jnp.where(jnp.abs(r - c) <= 2, 1.0, 0.0).astype(jnp.float32)


def _profile(hist, dim, nb):
    a = hist[:, :dim]
    mu = jnp.sum(a) / float(nb)
    var = jnp.sum((a - mu) ** 2) / float(nb - 1)
    sd = jnp.sqrt(var)
    a = jnp.clip(a, 0.0, mu + 3.0 * sd)
    a = a.astype(jnp.bfloat16).astype(jnp.float32)
    k = jnp.asarray(0.04, jnp.bfloat16).astype(jnp.float32)
    blur = k * jnp.dot(_band(_S), jnp.dot(a, _band(dim),
                                          preferred_element_type=jnp.float32, precision=lax.Precision.HIGHEST),
                       preferred_element_type=jnp.float32, precision=lax.Precision.HIGHEST)
    pos = lax.broadcasted_iota(jnp.int32, (_S, dim), 1).astype(jnp.float32)
    m = jnp.sum(blur * pos, axis=1, keepdims=True) / float(_SEGLEN)
    start = m[_SIDX:_SIDX + 1, :]
    aligned = (m - start) - (float(dim // 2) - start)
    return m, jnp.round(aligned).astype(jnp.int32)


def _outlier0(windows):
    med = _median10(windows)
    diffs = [jnp.abs(w - med) for w in windows]
    mad = _median10(diffs)
    safe = jnp.where(mad == 0.0, 1.0, mad)
    flagged = jnp.where((0.6745 * diffs[0] / safe) > 2.0, 1.0, 0.0)
    return jnp.where(mad == 0.0, 0.0, flagged)


def _stats_body(hx_ref, hy_ref, ax_ref, ay_ref, mask_ref):
    mx, ax = _profile(hx_ref[0], _W, _S * _W)
    my, ay = _profile(hy_ref[0], _H, _S * _H)

    wx = [mx[_SIDX + o:_EIDX + o, :] for o in range(10)]
    wy = [my[_SIDX + o:_EIDX + o, :] for o in range(10)]
    maskc = (1.0 - _outlier0(wx)) * (1.0 - _outlier0(wy))

    ax_ref[...] = ax[_SIDX:_EIDX, :].astype(jnp.float32).reshape(1, _NSEG, 1)
    ay_ref[...] = ay[_SIDX:_EIDX, :].astype(jnp.float32).reshape(1, _NSEG, 1)
    mask_ref[...] = maskc.reshape(1, _NSEG, 1)


def _stats_call(hx, hy):
    hist_spec = pl.BlockSpec((1, _S, _PAD), lambda b: (b, 0, 0))
    col_spec = pl.BlockSpec((1, _NSEG, 1), lambda b: (b, 0, 0))
    return pl.pallas_call(
        _stats_body,
        grid=(_B,),
        in_specs=[hist_spec, hist_spec],
        out_specs=[col_spec, col_spec, col_spec],
        out_shape=[jax.ShapeDtypeStruct((_B, _NSEG, 1), jnp.float32)] * 3,
    )(hx, hy)


def _dense_body(t_ref, x_ref, y_ref, tl_ref, ax_ref, ay_ref, mask_ref,
                w1_ref, b1_ref, w2_ref, b2_ref, w3_ref, b3_ref,
                idx_ref, vals_ref):
    th = t_ref[0] / tl_ref[0]
    ev0 = pl.program_id(1) * _CH
    seg = (ev0 + lax.broadcasted_iota(jnp.int32, (_CH, 1), 0)) // _SEGLEN
    segoff = (seg.astype(jnp.float32) + float(_SIDX)) / float(_EIDX - 1)
    tin = th - segoff

    h1 = _leaky(tin * w1_ref[...] + b1_ref[...])
    h2 = _leaky(jnp.dot(h1.astype(jnp.bfloat16), w2_ref[...].astype(jnp.bfloat16),
                        preferred_element_type=jnp.float32)
                + b2_ref[...])
    v = jnp.dot(h2.astype(jnp.bfloat16), w3_ref[...].astype(jnp.bfloat16),
                preferred_element_type=jnp.float32) + b3_ref[...]

    onehot = (seg == lax.broadcasted_iota(jnp.int32, (1, _NSEG), 1)) \
        .astype(jnp.float32)
    maskrep = jnp.dot(onehot, mask_ref[0], preferred_element_type=jnp.float32,
                      precision=lax.Precision.HIGHEST)
    vals = th * v * maskrep
    vals_ref[...] = vals.reshape(1, _CH, 1)

    def _coord(ref, al_ref, hi):
        al = jnp.dot(onehot, al_ref[0], preferred_element_type=jnp.float32,
                     precision=lax.Precision.HIGHEST).astype(jnp.int32)
        return jnp.clip(ref[0].astype(jnp.int32) - al, 0, hi - 1)

    xi = _coord(x_ref, ax_ref, _W)
    yi = _coord(y_ref, ay_ref, _H)
    slot = lax.rem(pl.program_id(0), 2)
    idx_ref[...] = (xi + _W * yi + slot * _HW).reshape(1, _CH, 1)


def _dense_call(hx, hy, t, x, y, w1, b1, w2, b2, w3, b3):
    ax, ay, maskc = _stats_call(hx, hy)
    full = lambda shape: pl.BlockSpec(shape, lambda b, c: (0,) * len(shape))
    seg_slice = lambda a: a[:, _SIDX * _SEGLEN:_EIDX * _SEGLEN].reshape(_B, _EV, 1)
    ev_spec = pl.BlockSpec((1, _CH, 1), lambda b, c: (b, c, 0))
    col_spec = pl.BlockSpec((1, _NSEG, 1), lambda b, c: (b, 0, 0))
    return pl.pallas_call(
        _dense_body,
        grid=(_B, _EV // _CH),
        in_specs=[
            ev_spec, ev_spec, ev_spec,
            pl.BlockSpec((1, 1, 1), lambda b, c: (b, 0, 0)),
            col_spec, col_spec, col_spec,
            full((1, 30)), full((1, 30)), full((30, 30)),
            full((1, 30)), full((30, 1)), full((1, 1)),
        ],
        out_specs=[ev_spec, ev_spec],
        out_shape=[
            jax.ShapeDtypeStruct((_B, _EV, 1), jnp.int32),
            jax.ShapeDtypeStruct((_B, _EV, 1), jnp.float32),
        ],
    )(seg_slice(t), seg_slice(x), seg_slice(y),
      t[:, _N - 1:].reshape(_B, 1, 1), ax, ay, maskc,
      w1, b1, w2, b2, w3, b3)


def kernel(events, w1, b1, w2, b2, w3, b3):
    x = events[..., 0]
    y = events[..., 1]
    t = events[..., 2]

    def _group(coord):
        g = coord.astype(jnp.int32).reshape(_B * _S // 16, 16, _SEGLEN)
        return g.transpose(0, 2, 1).reshape(_B * _S // 16, 16 * _SEGLEN)

    coords = jnp.concatenate([_group(x), _group(y)], axis=0)
    hists = _get_hist_kernel()(coords)
    hists = hists.reshape(2, _B, _S, _PAD)

    idx, vals = _dense_call(hists[0], hists[1], t, x, y,
                            w1, b1.reshape(1, 30), w2, b2.reshape(1, 30),
                            w3, b3.reshape(1, 1))

    idx4 = idx.reshape(_B, 8, _ROWS, 128)
    vals4 = vals.reshape(_B, 8, _ROWS, 128)
    out = _get_scatter_kernel()(idx4, vals4)
    return out.reshape(_B, _H, _W)


# speedup vs baseline: 22.2496x; 1.4653x over previous
import functools

import jax
import jax.numpy as jnp
from jax import lax
from jax.experimental import pallas as pl
from jax.experimental.pallas import tpu as pltpu
from jax.experimental.pallas import tpu_sc as plsc

_B, _N = 4, 49152
_H, _W = 180, 240
_S = 48
_SIDX, _ENDBIAS = 3, 10
_EIDX = _S - _ENDBIAS
_NSEG = _EIDX - _SIDX
_SEGLEN = _N // _S
_PAD = 256
_NGRP = 24
_HW = _H * _W
_EV = _NSEG * _SEGLEN
_CH = _EV // 8
_ROWS = _EV // 8 // 128
_ZCHUNK = 2 * _HW // 16
_ZBUF = 5408


def _hist_body(coords_hbm, out_hbm, coords_v, hist_v):
    wid = lax.axis_index("c") * 16 + lax.axis_index("s")

    @pl.when(wid < _NGRP)
    def _():
        pltpu.sync_copy(coords_hbm.at[wid], coords_v)
        zeros16 = jnp.zeros((16,), jnp.float32)

        def zero_body(i, carry):
            hist_v[pl.ds(i * 16, 16)] = zeros16
            return carry

        lax.fori_loop(0, _PAD, zero_body, 0)

        lane_base = lax.iota(jnp.int32, 16) * _PAD
        ones16 = jnp.ones((16,), jnp.float32)

        def ev_body(e, carry):
            v = coords_v[pl.ds(e * 16, 16)]
            plsc.addupdate_scatter(hist_v, [v + lane_base], ones16)
            return carry

        lax.fori_loop(0, _SEGLEN, ev_body, 0)
        pltpu.sync_copy(hist_v, out_hbm.at[wid])


@functools.lru_cache(maxsize=None)
def _get_hist_kernel():
    return pl.kernel(
        _hist_body,
        out_type=jax.ShapeDtypeStruct((_NGRP, 16 * _PAD), jnp.float32),
        mesh=plsc.VectorSubcoreMesh(core_axis_name="c", subcore_axis_name="s"),
        scratch_types=[
            pltpu.VMEM((_SEGLEN * 16,), jnp.int32),
            pltpu.VMEM((16 * _PAD,), jnp.float32),
        ],
        compiler_params=pltpu.CompilerParams(needs_layout_passes=False),
    )


def _scatter_body(idx_hbm, vals_hbm, out_hbm, idx_v, vals_v, zbuf_v, acc_s):
    c = lax.axis_index("c")
    s = lax.axis_index("s")

    zeros16 = jnp.zeros((16,), jnp.float32)

    def zero_body(i, carry):
        zbuf_v[pl.ds(i * 16, 16)] = zeros16
        return carry

    lax.fori_loop(0, _ZBUF // 16, zero_body, 0)
    pltpu.sync_copy(zbuf_v.at[pl.ds(0, _ZCHUNK)],
                    acc_s.at[pl.ds(s * _ZCHUNK, _ZCHUNK)])
    plsc.subcore_barrier()

    b = c * 2 + s // 8
    chunk = s % 8
    pltpu.sync_copy(idx_hbm.at[b, chunk], idx_v)
    pltpu.sync_copy(vals_hbm.at[b, chunk], vals_v)

    def row_body(j, carry):
        pltpu.sync_copy(vals_v.at[j], acc_s.at[idx_v.at[j]], add=True)
        return carry

    lax.fori_loop(0, _ROWS, row_body, 0)
    plsc.subcore_barrier()

    pltpu.sync_copy(acc_s.at[pl.ds(s * _ZCHUNK, _ZCHUNK)],
                    zbuf_v.at[pl.ds(0, _ZCHUNK)])
    pltpu.sync_copy(zbuf_v.at[pl.ds(0, _ZCHUNK)],
                    out_hbm.at[pl.ds(c * 2 * _HW + s * _ZCHUNK, _ZCHUNK)])


@functools.lru_cache(maxsize=None)
def _get_scatter_kernel():
    return pl.kernel(
        _scatter_body,
        out_type=jax.ShapeDtypeStruct((_B * _HW,), jnp.float32),
        mesh=plsc.VectorSubcoreMesh(core_axis_name="c", subcore_axis_name="s"),
        scratch_types=[
            pltpu.VMEM((_ROWS, 128), jnp.int32),
            pltpu.VMEM((_ROWS, 128), jnp.float32),
            pltpu.VMEM((_ZBUF,), jnp.float32),
            pltpu.VMEM_SHARED((2 * _HW,), jnp.float32),
        ],
        compiler_params=pltpu.CompilerParams(needs_layout_passes=False),
    )


def _leaky(x):
    return jnp.where(x >= 0, x, 0.1 * x)


def _sortnet10(vs):
    vs = list(vs)
    for r in range(10):
        for i in range(r % 2, 9, 2):
            lo = jnp.minimum(vs[i], vs[i + 1])
            hi = jnp.maximum(vs[i], vs[i + 1])
            vs[i], vs[i + 1] = lo, hi
    return vs


def _median10(vs):
    sv = _sortnet10(vs)
    return 0.5 * (sv[4] + sv[5])


def _band(n):
    r = lax.broadcasted_iota(jnp.int32, (n, n), 0)
    c = lax.broadcasted_iota(jnp.int32, (n, n), 1)
    return jnp.where(jnp.abs(r - c) <= 2, 1.0, 0.0).astype(jnp.float32)


def _profile(hist, dim, nb):
    a = hist[:, :dim]
    mu = jnp.sum(a) / float(nb)
    var = jnp.sum((a - mu) ** 2) / float(nb - 1)
    sd = jnp.sqrt(var)
    a = jnp.clip(a, 0.0, mu + 3.0 * sd)
    ab = a.astype(jnp.bfloat16)
    k = jnp.asarray(0.04, jnp.bfloat16).astype(jnp.float32)
    row = jnp.dot(ab, _band(dim).astype(jnp.bfloat16),
                  preferred_element_type=jnp.float32)
    zpad = jnp.zeros((2, dim), jnp.float32)
    rp = jnp.concatenate([zpad, row, zpad], axis=0)
    blur = k * (rp[0:_S] + rp[1:_S + 1] + rp[2:_S + 2]
                + rp[3:_S + 3] + rp[4:_S + 4])
    pos = lax.broadcasted_iota(jnp.int32, (_S, dim), 1).astype(jnp.float32)
    m = jnp.sum(blur * pos, axis=1, keepdims=True) / float(_SEGLEN)
    start = m[_SIDX:_SIDX + 1, :]
    aligned = (m - start) - (float(dim // 2) - start)
    return m, jnp.round(aligned).astype(jnp.int32)


def _outlier0(windows):
    med = _median10(windows)
    diffs = [jnp.abs(w - med) for w in windows]
    mad = _median10(diffs)
    safe = jnp.where(mad == 0.0, 1.0, mad)
    flagged = jnp.where((0.6745 * diffs[0] / safe) > 2.0, 1.0, 0.0)
    return jnp.where(mad == 0.0, 0.0, flagged)


def _stats_body(hx_ref, hy_ref, cols_ref):
    mx, ax = _profile(hx_ref[0], _W, _S * _W)
    my, ay = _profile(hy_ref[0], _H, _S * _H)

    wx = [mx[_SIDX + o:_EIDX + o, :] for o in range(10)]
    wy = [my[_SIDX + o:_EIDX + o, :] for o in range(10)]
    maskc = (1.0 - _outlier0(wx)) * (1.0 - _outlier0(wy))

    cols = jnp.concatenate(
        [maskc, ax[_SIDX:_EIDX, :].astype(jnp.float32),
         ay[_SIDX:_EIDX, :].astype(jnp.float32)], axis=1)
    cols_ref[...] = cols.reshape(1, _NSEG, 3)


def _stats_call(hx, hy):
    hist_spec = pl.BlockSpec((1, _S, _PAD), lambda b: (b, 0, 0))
    return pl.pallas_call(
        _stats_body,
        grid=(_B,),
        in_specs=[hist_spec, hist_spec],
        out_specs=pl.BlockSpec((1, _NSEG, 3), lambda b: (b, 0, 0)),
        out_shape=jax.ShapeDtypeStruct((_B, _NSEG, 3), jnp.float32),
    )(hx, hy)


def _dense_body(t_ref, x_ref, y_ref, tl_ref, cols_ref,
                w1_ref, b1_ref, w2_ref, b2_ref, w3_ref, b3_ref,
                idx_ref, vals_ref):
    th = t_ref[0] / tl_ref[0]
    ev0 = pl.program_id(1) * _CH
    seg = (ev0 + lax.broadcasted_iota(jnp.int32, (_CH, 1), 0)) // _SEGLEN
    segoff = (seg.astype(jnp.float32) + float(_SIDX)) / float(_EIDX - 1)
    tin = th - segoff

    h1 = _leaky(tin * w1_ref[...] + b1_ref[...])
    h2 = _leaky(jnp.dot(h1.astype(jnp.bfloat16), w2_ref[...].astype(jnp.bfloat16),
                        preferred_element_type=jnp.float32)
                + b2_ref[...])
    v = jnp.dot(h2.astype(jnp.bfloat16), w3_ref[...].astype(jnp.bfloat16),
                preferred_element_type=jnp.float32) + b3_ref[...]

    onehot = (seg == lax.broadcasted_iota(jnp.int32, (1, _NSEG), 1)) \
        .astype(jnp.bfloat16)
    rep = jnp.dot(onehot, cols_ref[0].astype(jnp.bfloat16),
                  preferred_element_type=jnp.float32)
    vals = th * v * rep[:, 0:1]
    vals_ref[...] = vals.reshape(1, _CH, 1)

    xi = jnp.clip(x_ref[0].astype(jnp.int32) - rep[:, 1:2].astype(jnp.int32),
                  0, _W - 1)
    yi = jnp.clip(y_ref[0].astype(jnp.int32) - rep[:, 2:3].astype(jnp.int32),
                  0, _H - 1)
    slot = lax.rem(pl.program_id(0), 2)
    idx_ref[...] = (xi + _W * yi + slot * _HW).reshape(1, _CH, 1)


def _dense_call(hx, hy, t, x, y, w1, b1, w2, b2, w3, b3):
    cols = _stats_call(hx, hy)
    full = lambda shape: pl.BlockSpec(shape, lambda b, c: (0,) * len(shape))
    seg_slice = lambda a: a[:, _SIDX * _SEGLEN:_EIDX * _SEGLEN].reshape(_B, _EV, 1)
    ev_spec = pl.BlockSpec((1, _CH, 1), lambda b, c: (b, c, 0))
    return pl.pallas_call(
        _dense_body,
        grid=(_B, _EV // _CH),
        in_specs=[
            ev_spec, ev_spec, ev_spec,
            pl.BlockSpec((1, 1, 1), lambda b, c: (b, 0, 0)),
            pl.BlockSpec((1, _NSEG, 3), lambda b, c: (b, 0, 0)),
            full((1, 30)), full((1, 30)), full((30, 30)),
            full((1, 30)), full((30, 1)), full((1, 1)),
        ],
        out_specs=[ev_spec, ev_spec],
        out_shape=[
            jax.ShapeDtypeStruct((_B, _EV, 1), jnp.int32),
            jax.ShapeDtypeStruct((_B, _EV, 1), jnp.float32),
        ],
    )(seg_slice(t), seg_slice(x), seg_slice(y),
      t[:, _N - 1:].reshape(_B, 1, 1), cols,
      w1, b1, w2, b2, w3, b3)


def kernel(events, w1, b1, w2, b2, w3, b3):
    x = events[..., 0]
    y = events[..., 1]
    t = events[..., 2]

    def _group(coord):
        g = coord.astype(jnp.int32).reshape(_B * _S // 16, 16, _SEGLEN)
        return g.transpose(0, 2, 1).reshape(_B * _S // 16, 16 * _SEGLEN)

    coords = jnp.concatenate([_group(x), _group(y)], axis=0)
    hists = _get_hist_kernel()(coords)
    hists = hists.reshape(2, _B, _S, _PAD)

    idx, vals = _dense_call(hists[0], hists[1], t, x, y,
                            w1, b1.reshape(1, 30), w2, b2.reshape(1, 30),
                            w3, b3.reshape(1, 1))

    idx4 = idx.reshape(_B, 8, _ROWS, 128)
    vals4 = vals.reshape(_B, 8, _ROWS, 128)
    out = _get_scatter_kernel()(idx4, vals4)
    return out.reshape(_B, _H, _W)


# speedup vs baseline: 41.4701x; 1.8639x over previous
import functools

import jax
import jax.numpy as jnp
from jax import lax
from jax.experimental import pallas as pl
from jax.experimental.pallas import tpu as pltpu
from jax.experimental.pallas import tpu_sc as plsc

_B, _N = 4, 49152
_H, _W = 180, 240
_S = 48
_SIDX, _ENDBIAS = 3, 10
_EIDX = _S - _ENDBIAS
_NSEG = _EIDX - _SIDX
_SEGLEN = _N // _S
_PAD = 256
_NGRP = 24
_HW = _H * _W
_EV = _NSEG * _SEGLEN
_CSEG = 7
_CH = _CSEG * _SEGLEN
_ROWS = _EV // 8 // 128
_ZCHUNK = 2 * _HW // 16
_ZBUF = 5408


def _hist_body(coords_hbm, out_hbm, coords_v, hist_v):
    wid = lax.axis_index("c") * 16 + lax.axis_index("s")

    @pl.when(wid < _NGRP)
    def _():
        pltpu.sync_copy(coords_hbm.at[wid], coords_v)
        zeros16 = jnp.zeros((16,), jnp.float32)

        def zero_body(i, carry):
            hist_v[pl.ds(i * 16, 16)] = zeros16
            return carry

        lax.fori_loop(0, _PAD, zero_body, 0)

        lane_base = lax.iota(jnp.int32, 16) * _PAD
        ones16 = jnp.ones((16,), jnp.float32)

        def ev_body(e, carry):
            v = coords_v[pl.ds(e * 16, 16)]
            plsc.addupdate_scatter(hist_v, [v + lane_base], ones16)
            return carry

        lax.fori_loop(0, _SEGLEN, ev_body, 0)
        pltpu.sync_copy(hist_v, out_hbm.at[wid])


@functools.lru_cache(maxsize=None)
def _get_hist_kernel():
    return pl.kernel(
        _hist_body,
        out_type=jax.ShapeDtypeStruct((_NGRP, 16 * _PAD), jnp.float32),
        mesh=plsc.VectorSubcoreMesh(core_axis_name="c", subcore_axis_name="s"),
        scratch_types=[
            pltpu.VMEM((_SEGLEN * 16,), jnp.int32),
            pltpu.VMEM((16 * _PAD,), jnp.float32),
        ],
        compiler_params=pltpu.CompilerParams(needs_layout_passes=False),
    )


def _scatter_body(idx_hbm, vals_hbm, out_hbm, idx_v, vals_v, zbuf_v, acc_s):
    c = lax.axis_index("c")
    s = lax.axis_index("s")

    zeros16 = jnp.zeros((16,), jnp.float32)

    def zero_body(i, carry):
        zbuf_v[pl.ds(i * 16, 16)] = zeros16
        return carry

    lax.fori_loop(0, _ZBUF // 16, zero_body, 0)
    pltpu.sync_copy(zbuf_v.at[pl.ds(0, _ZCHUNK)],
                    acc_s.at[pl.ds(s * _ZCHUNK, _ZCHUNK)])
    plsc.subcore_barrier()

    b = c * 2 + s // 8
    chunk = s % 8
    pltpu.sync_copy(idx_hbm.at[b, chunk], idx_v)
    pltpu.sync_copy(vals_hbm.at[b, chunk], vals_v)

    def row_body(j, carry):
        pltpu.sync_copy(vals_v.at[j], acc_s.at[idx_v.at[j]], add=True)
        return carry

    lax.fori_loop(0, _ROWS, row_body, 0)
    plsc.subcore_barrier()

    pltpu.sync_copy(acc_s.at[pl.ds(s * _ZCHUNK, _ZCHUNK)],
                    zbuf_v.at[pl.ds(0, _ZCHUNK)])
    pltpu.sync_copy(zbuf_v.at[pl.ds(0, _ZCHUNK)],
                    out_hbm.at[pl.ds(c * 2 * _HW + s * _ZCHUNK, _ZCHUNK)])


@functools.lru_cache(maxsize=None)
def _get_scatter_kernel():
    return pl.kernel(
        _scatter_body,
        out_type=jax.ShapeDtypeStruct((_B * _HW,), jnp.float32),
        mesh=plsc.VectorSubcoreMesh(core_axis_name="c", subcore_axis_name="s"),
        scratch_types=[
            pltpu.VMEM((_ROWS, 128), jnp.int32),
            pltpu.VMEM((_ROWS, 128), jnp.float32),
            pltpu.VMEM((_ZBUF,), jnp.float32),
            pltpu.VMEM_SHARED((2 * _HW,), jnp.float32),
        ],
        compiler_params=pltpu.CompilerParams(needs_layout_passes=False),
    )


def _leaky(x):
    return jnp.where(x >= 0, x, 0.1 * x)


def _sortnet10(vs):
    vs = list(vs)
    for r in range(10):
        for i in range(r % 2, 9, 2):
            lo = jnp.minimum(vs[i], vs[i + 1])
            hi = jnp.maximum(vs[i], vs[i + 1])
            vs[i], vs[i + 1] = lo, hi
    return vs


def _median10(vs):
    sv = _sortnet10(vs)
    return 0.5 * (sv[4] + sv[5])


def _band(n):
    r = lax.broadcasted_iota(jnp.int32, (n, n), 0)
    c = lax.broadcasted_iota(jnp.int32, (n, n), 1)
    return jnp.where(jnp.abs(r - c) <= 2, 1.0, 0.0).astype(jnp.float32)


def _profile(hist, dim, nb):
    a = hist[:, :dim]
    mu = jnp.sum(a) / float(nb)
    var = jnp.sum((a - mu) ** 2) / float(nb - 1)
    sd = jnp.sqrt(var)
    a = jnp.clip(a, 0.0, mu + 3.0 * sd)
    ab = a.astype(jnp.bfloat16)
    k = jnp.asarray(0.04, jnp.bfloat16).astype(jnp.float32)
    row = jnp.dot(ab, _band(dim).astype(jnp.bfloat16),
                  preferred_element_type=jnp.float32)
    zpad = jnp.zeros((2, dim), jnp.float32)
    rp = jnp.concatenate([zpad, row, zpad], axis=0)
    blur = k * (rp[0:_S] + rp[1:_S + 1] + rp[2:_S + 2]
                + rp[3:_S + 3] + rp[4:_S + 4])
    pos = lax.broadcasted_iota(jnp.int32, (_S, dim), 1).astype(jnp.float32)
    m = jnp.sum(blur * pos, axis=1, keepdims=True) / float(_SEGLEN)
    start = m[_SIDX:_SIDX + 1, :]
    aligned = (m - start) - (float(dim // 2) - start)
    return m, jnp.round(aligned).astype(jnp.int32)


def _outlier0(windows):
    med = _median10(windows)
    diffs = [jnp.abs(w - med) for w in windows]
    mad = _median10(diffs)
    safe = jnp.where(mad == 0.0, 1.0, mad)
    flagged = jnp.where((0.6745 * diffs[0] / safe) > 2.0, 1.0, 0.0)
    return jnp.where(mad == 0.0, 0.0, flagged)


def _stats_body(hx_ref, hy_ref, cols_ref):
    mx, ax = _profile(hx_ref[0], _W, _S * _W)
    my, ay = _profile(hy_ref[0], _H, _S * _H)

    wx = [mx[_SIDX + o:_EIDX + o, :] for o in range(10)]
    wy = [my[_SIDX + o:_EIDX + o, :] for o in range(10)]
    maskc = (1.0 - _outlier0(wx)) * (1.0 - _outlier0(wy))

    cols = jnp.concatenate(
        [maskc, ax[_SIDX:_EIDX, :].astype(jnp.float32),
         ay[_SIDX:_EIDX, :].astype(jnp.float32)], axis=1)
    cols_ref[...] = cols.reshape(1, _NSEG, 3)


def _stats_call(hx, hy):
    hist_spec = pl.BlockSpec((1, _S, _PAD), lambda b: (b, 0, 0))
    return pl.pallas_call(
        _stats_body,
        grid=(_B,),
        in_specs=[hist_spec, hist_spec],
        out_specs=pl.BlockSpec((1, _NSEG, 3), lambda b: (b, 0, 0)),
        out_shape=jax.ShapeDtypeStruct((_B, _NSEG, 3), jnp.float32),
    )(hx, hy)


def _idx_body(x_ref, y_ref, cols_ref, idx_ref):
    def rep(col):
        c3 = col.reshape(_NSEG, 1, 1).astype(jnp.int32)
        return jnp.broadcast_to(c3, (_NSEG, 8, 128)).reshape(_NSEG * 8, 128)

    cols = cols_ref[0]
    xi = jnp.clip(x_ref[0].astype(jnp.int32) - rep(cols[:, 1:2]), 0, _W - 1)
    yi = jnp.clip(y_ref[0].astype(jnp.int32) - rep(cols[:, 2:3]), 0, _H - 1)
    slot = lax.rem(pl.program_id(0), 2)
    idx_ref[...] = (xi + _W * yi + slot * _HW).reshape(1, _NSEG * 8, 128)


def _vals_body(t_ref, tl_ref, mk_ref, w1_ref, b1_ref, w2_ref, b2_ref,
               w3_ref, b3_ref, vals_ref):
    th = t_ref[0] / tl_ref[0]
    c = pl.program_id(1)
    s3 = lax.broadcasted_iota(jnp.int32, (_CSEG, 1, 1), 0) \
        + c * _CSEG + _SIDX
    segoff = (s3.astype(jnp.float32) / float(_EIDX - 1))
    tin = th - jnp.broadcast_to(segoff, (_CSEG, _SEGLEN, 1)).reshape(_CH, 1)

    h1 = _leaky(tin * w1_ref[...] + b1_ref[...])
    h2 = _leaky(jnp.dot(h1.astype(jnp.bfloat16), w2_ref[...].astype(jnp.bfloat16),
                        preferred_element_type=jnp.float32)
                + b2_ref[...])
    v = jnp.dot(h2.astype(jnp.bfloat16), w3_ref[...].astype(jnp.bfloat16),
                preferred_element_type=jnp.float32) + b3_ref[...]

    mrep = jnp.broadcast_to(mk_ref[0, 0].reshape(_CSEG, 1, 1),
                            (_CSEG, _SEGLEN, 1)).reshape(_CH, 1)
    vals_ref[...] = (th * v * mrep).reshape(1, _CH, 1)


def _dense_call(hx, hy, t, x, y, w1, b1, w2, b2, w3, b3):
    cols = _stats_call(hx, hy)
    seg_slice = lambda a: a[:, _SIDX * _SEGLEN:_EIDX * _SEGLEN]
    rows = _NSEG * 8

    idx = pl.pallas_call(
        _idx_body,
        grid=(_B,),
        in_specs=[
            pl.BlockSpec((1, rows, 128), lambda b: (b, 0, 0)),
            pl.BlockSpec((1, rows, 128), lambda b: (b, 0, 0)),
            pl.BlockSpec((1, _NSEG, 3), lambda b: (b, 0, 0)),
        ],
        out_specs=pl.BlockSpec((1, rows, 128), lambda b: (b, 0, 0)),
        out_shape=jax.ShapeDtypeStruct((_B, rows, 128), jnp.int32),
    )(seg_slice(x).reshape(_B, rows, 128), seg_slice(y).reshape(_B, rows, 128),
      cols)

    full = lambda shape: pl.BlockSpec(shape, lambda b, c: (0,) * len(shape))
    vals = pl.pallas_call(
        _vals_body,
        grid=(_B, _NSEG // _CSEG),
        in_specs=[
            pl.BlockSpec((1, _CH, 1), lambda b, c: (b, c, 0)),
            pl.BlockSpec((1, 1, 1), lambda b, c: (b, 0, 0)),
            pl.BlockSpec((1, 1, _CSEG, 1), lambda b, c: (b, c, 0, 0)),
            full((1, 30)), full((1, 30)), full((30, 30)),
            full((1, 30)), full((30, 1)), full((1, 1)),
        ],
        out_specs=pl.BlockSpec((1, _CH, 1), lambda b, c: (b, c, 0)),
        out_shape=jax.ShapeDtypeStruct((_B, _EV, 1), jnp.float32),
    )(seg_slice(t).reshape(_B, _EV, 1), t[:, _N - 1:].reshape(_B, 1, 1),
      cols[:, :, 0:1].reshape(_B, _NSEG // _CSEG, _CSEG, 1), w1, b1, w2, b2, w3, b3)
    return idx, vals


def kernel(events, w1, b1, w2, b2, w3, b3):
    x = events[..., 0]
    y = events[..., 1]
    t = events[..., 2]

    def _group(coord):
        g = coord.astype(jnp.int32).reshape(_B * _S // 16, 16, _SEGLEN)
        return g.transpose(0, 2, 1).reshape(_B * _S // 16, 16 * _SEGLEN)

    coords = jnp.concatenate([_group(x), _group(y)], axis=0)
    hists = _get_hist_kernel()(coords)
    hists = hists.reshape(2, _B, _S, _PAD)

    idx, vals = _dense_call(hists[0], hists[1], t, x, y,
                            w1, b1.reshape(1, 30), w2, b2.reshape(1, 30),
                            w3, b3.reshape(1, 1))

    idx4 = idx.reshape(_B, 8, _ROWS, 128)
    vals4 = vals.reshape(_B, 8, _ROWS, 128)
    out = _get_scatter_kernel()(idx4, vals4)
    return out.reshape(_B, _H, _W)


# speedup vs baseline: 42.2658x; 1.0192x over previous
import functools

import jax
import jax.numpy as jnp
from jax import lax
from jax.experimental import pallas as pl
from jax.experimental.pallas import tpu as pltpu
from jax.experimental.pallas import tpu_sc as plsc

_B, _N = 4, 49152
_H, _W = 180, 240
_S = 48
_SIDX, _ENDBIAS = 3, 10
_EIDX = _S - _ENDBIAS
_NSEG = _EIDX - _SIDX
_SEGLEN = _N // _S
_PAD = 256
_NGRP = 24
_HW = _H * _W
_EV = _NSEG * _SEGLEN
_CSEG = 7
_CH = _CSEG * _SEGLEN
_ROWS = _EV // 8 // 128
_ZCHUNK = 2 * _HW // 16
_ZBUF = 5408


def _hist_body(coords_hbm, out_hbm, coords_v, hist_v):
    wid = lax.axis_index("c") * 16 + lax.axis_index("s")

    @pl.when(wid < _NGRP)
    def _():
        pltpu.sync_copy(coords_hbm.at[wid], coords_v)
        zeros16 = jnp.zeros((16,), jnp.float32)

        def zero_body(i, carry):
            hist_v[pl.ds(i * 16, 16)] = zeros16
            return carry

        lax.fori_loop(0, _PAD, zero_body, 0)

        lane_base = lax.iota(jnp.int32, 16) * _PAD
        ones16 = jnp.ones((16,), jnp.float32)

        def ev_body(e, carry):
            v = coords_v[pl.ds(e * 16, 16)]
            plsc.addupdate_scatter(hist_v, [v + lane_base], ones16)
            return carry

        lax.fori_loop(0, _SEGLEN, ev_body, 0)
        pltpu.sync_copy(hist_v, out_hbm.at[wid])


@functools.lru_cache(maxsize=None)
def _get_hist_kernel():
    return pl.kernel(
        _hist_body,
        out_type=jax.ShapeDtypeStruct((_NGRP, 16 * _PAD), jnp.float32),
        mesh=plsc.VectorSubcoreMesh(core_axis_name="c", subcore_axis_name="s"),
        scratch_types=[
            pltpu.VMEM((_SEGLEN * 16,), jnp.int32),
            pltpu.VMEM((16 * _PAD,), jnp.float32),
        ],
        compiler_params=pltpu.CompilerParams(needs_layout_passes=False),
    )


def _scatter_body(idx_hbm, vals_hbm, out_hbm, idx_v, vals_v, zbuf_v, acc_s):
    c = lax.axis_index("c")
    s = lax.axis_index("s")

    zeros16 = jnp.zeros((16,), jnp.float32)

    def zero_body(i, carry):
        zbuf_v[pl.ds(i * 16, 16)] = zeros16
        return carry

    lax.fori_loop(0, _ZBUF // 16, zero_body, 0)
    pltpu.sync_copy(zbuf_v.at[pl.ds(0, _ZCHUNK)],
                    acc_s.at[pl.ds(s * _ZCHUNK, _ZCHUNK)])
    plsc.subcore_barrier()

    b = c * 2 + s // 8
    chunk = s % 8
    pltpu.sync_copy(idx_hbm.at[b, chunk], idx_v)
    pltpu.sync_copy(vals_hbm.at[b, chunk], vals_v)

    def row_body(j, carry):
        pltpu.sync_copy(vals_v.at[j], acc_s.at[idx_v.at[j]], add=True)
        return carry

    lax.fori_loop(0, _ROWS, row_body, 0)
    plsc.subcore_barrier()

    pltpu.sync_copy(acc_s.at[pl.ds(s * _ZCHUNK, _ZCHUNK)],
                    zbuf_v.at[pl.ds(0, _ZCHUNK)])
    pltpu.sync_copy(zbuf_v.at[pl.ds(0, _ZCHUNK)],
                    out_hbm.at[pl.ds(c * 2 * _HW + s * _ZCHUNK, _ZCHUNK)])


@functools.lru_cache(maxsize=None)
def _get_scatter_kernel():
    return pl.kernel(
        _scatter_body,
        out_type=jax.ShapeDtypeStruct((_B * _HW,), jnp.float32),
        mesh=plsc.VectorSubcoreMesh(core_axis_name="c", subcore_axis_name="s"),
        scratch_types=[
            pltpu.VMEM((_ROWS, 128), jnp.int32),
            pltpu.VMEM((_ROWS, 128), jnp.float32),
            pltpu.VMEM((_ZBUF,), jnp.float32),
            pltpu.VMEM_SHARED((2 * _HW,), jnp.float32),
        ],
        compiler_params=pltpu.CompilerParams(needs_layout_passes=False),
    )


def _leaky(x):
    return jnp.where(x >= 0, x, 0.1 * x)


def _sortnet10(vs):
    vs = list(vs)
    for r in range(10):
        for i in range(r % 2, 9, 2):
            lo = jnp.minimum(vs[i], vs[i + 1])
            hi = jnp.maximum(vs[i], vs[i + 1])
            vs[i], vs[i + 1] = lo, hi
    return vs


def _median10(vs):
    sv = _sortnet10(vs)
    return 0.5 * (sv[4] + sv[5])


def _band(n):
    r = lax.broadcasted_iota(jnp.int32, (n, n), 0)
    c = lax.broadcasted_iota(jnp.int32, (n, n), 1)
    return jnp.where(jnp.abs(r - c) <= 2, 1.0, 0.0).astype(jnp.float32)


def _profile(hist, dim, nb):
    a = hist[:, :dim]
    mu = jnp.sum(a) / float(nb)
    var = jnp.sum((a - mu) ** 2) / float(nb - 1)
    sd = jnp.sqrt(var)
    a = jnp.clip(a, 0.0, mu + 3.0 * sd)
    ab = a.astype(jnp.bfloat16)
    k = jnp.asarray(0.04, jnp.bfloat16).astype(jnp.float32)
    row = jnp.dot(ab, _band(dim).astype(jnp.bfloat16),
                  preferred_element_type=jnp.float32)
    zpad = jnp.zeros((2, dim), jnp.float32)
    rp = jnp.concatenate([zpad, row, zpad], axis=0)
    blur = k * (rp[0:_S] + rp[1:_S + 1] + rp[2:_S + 2]
                + rp[3:_S + 3] + rp[4:_S + 4])
    pos = lax.broadcasted_iota(jnp.int32, (_S, dim), 1).astype(jnp.float32)
    m = jnp.sum(blur * pos, axis=1, keepdims=True) / float(_SEGLEN)
    start = m[_SIDX:_SIDX + 1, :]
    aligned = (m - start) - (float(dim // 2) - start)
    return m, jnp.round(aligned).astype(jnp.int32)


def _outlier0(windows):
    med = _median10(windows)
    diffs = [jnp.abs(w - med) for w in windows]
    mad = _median10(diffs)
    safe = jnp.where(mad == 0.0, 1.0, mad)
    flagged = jnp.where((0.6745 * diffs[0] / safe) > 2.0, 1.0, 0.0)
    return jnp.where(mad == 0.0, 0.0, flagged)


def _stats_body(hx_ref, hy_ref, cols_ref):
    mx, ax = _profile(hx_ref[0], _W, _S * _W)
    my, ay = _profile(hy_ref[0], _H, _S * _H)

    wx = [mx[_SIDX + o:_EIDX + o, :] for o in range(10)]
    wy = [my[_SIDX + o:_EIDX + o, :] for o in range(10)]
    maskc = (1.0 - _outlier0(wx)) * (1.0 - _outlier0(wy))

    cols = jnp.concatenate(
        [maskc, ax[_SIDX:_EIDX, :].astype(jnp.float32),
         ay[_SIDX:_EIDX, :].astype(jnp.float32)], axis=1)
    cols_ref[...] = cols.reshape(1, _NSEG, 3)


def _stats_call(hx, hy):
    hist_spec = pl.BlockSpec((1, _S, _PAD), lambda b: (b, 0, 0))
    return pl.pallas_call(
        _stats_body,
        grid=(_B,),
        in_specs=[hist_spec, hist_spec],
        out_specs=pl.BlockSpec((1, _NSEG, 3), lambda b: (b, 0, 0)),
        out_shape=jax.ShapeDtypeStruct((_B, _NSEG, 3), jnp.float32),
    )(hx, hy)


def _ev_body(t_ref, x_ref, y_ref, tl_ref, cols_ref, w1_ref, b1_ref,
             w2_ref, b2_ref, w3_ref, b3_ref, idx_ref, vals_ref):
    th = t_ref[0] / tl_ref[0]
    c = pl.program_id(1)
    s3 = lax.broadcasted_iota(jnp.int32, (_CSEG, 1, 1), 0) \
        + c * _CSEG + _SIDX
    segoff = (s3.astype(jnp.float32) / float(_EIDX - 1))
    tin = th - jnp.broadcast_to(segoff, (_CSEG, _SEGLEN, 1)).reshape(_CH, 1)

    h1 = _leaky(tin * w1_ref[...] + b1_ref[...])
    h2 = _leaky(jnp.dot(h1.astype(jnp.bfloat16), w2_ref[...].astype(jnp.bfloat16),
                        preferred_element_type=jnp.float32)
                + b2_ref[...])
    v = jnp.dot(h2.astype(jnp.bfloat16), w3_ref[...].astype(jnp.bfloat16),
                preferred_element_type=jnp.float32) + b3_ref[...]

    cols = cols_ref[0, 0]
    mrep = jnp.broadcast_to(cols[:, 0:1].reshape(_CSEG, 1, 1),
                            (_CSEG, _SEGLEN, 1)).reshape(_CH, 1)
    vals_ref[...] = (th * v * mrep).reshape(1, _CH, 1)

    def rep(col):
        c3 = col.reshape(_CSEG, 1, 1).astype(jnp.int32)
        return jnp.broadcast_to(c3, (_CSEG, 8, 128)).reshape(_CSEG * 8, 128)

    xi = jnp.clip(x_ref[0].astype(jnp.int32) - rep(cols[:, 1:2]), 0, _W - 1)
    yi = jnp.clip(y_ref[0].astype(jnp.int32) - rep(cols[:, 2:3]), 0, _H - 1)
    slot = lax.rem(pl.program_id(0), 2)
    idx_ref[...] = (xi + _W * yi + slot * _HW).reshape(1, _CSEG * 8, 128)


def _dense_call(hx, hy, t, x, y, w1, b1, w2, b2, w3, b3):
    cols = _stats_call(hx, hy)
    seg_slice = lambda a: a[:, _SIDX * _SEGLEN:_EIDX * _SEGLEN]
    rows = _NSEG * 8
    crows = _CSEG * 8
    nch = _NSEG // _CSEG

    full = lambda shape: pl.BlockSpec(shape, lambda b, c: (0,) * len(shape))
    idx, vals = pl.pallas_call(
        _ev_body,
        grid=(_B, nch),
        in_specs=[
            pl.BlockSpec((1, _CH, 1), lambda b, c: (b, c, 0)),
            pl.BlockSpec((1, crows, 128), lambda b, c: (b, c, 0)),
            pl.BlockSpec((1, crows, 128), lambda b, c: (b, c, 0)),
            pl.BlockSpec((1, 1, 1), lambda b, c: (b, 0, 0)),
            pl.BlockSpec((1, 1, _CSEG, 3), lambda b, c: (b, c, 0, 0)),
            full((1, 30)), full((1, 30)), full((30, 30)),
            full((1, 30)), full((30, 1)), full((1, 1)),
        ],
        out_specs=[
            pl.BlockSpec((1, crows, 128), lambda b, c: (b, c, 0)),
            pl.BlockSpec((1, _CH, 1), lambda b, c: (b, c, 0)),
        ],
        out_shape=[
            jax.ShapeDtypeStruct((_B, rows, 128), jnp.int32),
            jax.ShapeDtypeStruct((_B, _EV, 1), jnp.float32),
        ],
    )(seg_slice(t).reshape(_B, _EV, 1),
      seg_slice(x).reshape(_B, rows, 128), seg_slice(y).reshape(_B, rows, 128),
      t[:, _N - 1:].reshape(_B, 1, 1),
      cols.reshape(_B, nch, _CSEG, 3), w1, b1, w2, b2, w3, b3)
    return idx, vals


def kernel(events, w1, b1, w2, b2, w3, b3):
    x = events[..., 0]
    y = events[..., 1]
    t = events[..., 2]

    def _group(coord):
        g = coord.astype(jnp.int32).reshape(_B * _S // 16, 16, _SEGLEN)
        return g.transpose(0, 2, 1).reshape(_B * _S // 16, 16 * _SEGLEN)

    coords = jnp.concatenate([_group(x), _group(y)], axis=0)
    hists = _get_hist_kernel()(coords)
    hists = hists.reshape(2, _B, _S, _PAD)

    idx, vals = _dense_call(hists[0], hists[1], t, x, y,
                            w1, b1.reshape(1, 30), w2, b2.reshape(1, 30),
                            w3, b3.reshape(1, 1))

    idx4 = idx.reshape(_B, 8, _ROWS, 128)
    vals4 = vals.reshape(_B, 8, _ROWS, 128)
    out = _get_scatter_kernel()(idx4, vals4)
    return out.reshape(_B, _H, _W)
